# Initial kernel scaffold; baseline (speedup 1.0000x reference)
import jax
import jax.numpy as jnp
from jax.experimental import pallas as pl


def kernel(input, table):
    raise NotImplementedError("write your pallas kernel here")


# speedup vs baseline: 1.1072x; 1.1072x over previous
import functools

import jax
import jax.numpy as jnp
from jax import lax
from jax.experimental import pallas as pl
from jax.experimental.pallas import tpu as pltpu
from jax.experimental.pallas import tpu_sc as plsc


@functools.lru_cache(maxsize=None)
def _make_gather(V, D, B):
    info = plsc.get_sparse_core_info()
    NC, NS = info.num_cores, info.num_subcores
    NW = NC * NS
    assert B % NW == 0
    b_per_w = B // NW
    C = 1600
    while b_per_w % C != 0:
        C //= 2
    n_chunks = b_per_w // C
    mesh = plsc.VectorSubcoreMesh(core_axis_name="c", subcore_axis_name="s")

    @functools.partial(
        pl.kernel,
        mesh=mesh,
        out_type=jax.ShapeDtypeStruct((B, D), jnp.float32),
        scratch_types=[
            pltpu.VMEM((C,), jnp.int32),
            pltpu.VMEM((C, D), jnp.float32),
            pltpu.SemaphoreType.DMA,
        ],
        compiler_params=pltpu.CompilerParams(use_tc_tiling_on_sc=False),
    )
    def gather(table_hbm, idx_hbm, out_hbm, idx_v, rows_v, sem):
        wid = lax.axis_index("s") * NC + lax.axis_index("c")
        base = wid * b_per_w

        def body(g, carry):
            off = base + g * C
            pltpu.sync_copy(idx_hbm.at[pl.ds(off, C)], idx_v)
            pltpu.async_copy(table_hbm.at[idx_v], rows_v, sem).wait()
            pltpu.sync_copy(rows_v, out_hbm.at[pl.ds(off, C)])
            return carry

        lax.fori_loop(0, n_chunks, body, 0)

    return gather


def kernel(input, table):
    T, Bt = input.shape
    V, D = table.shape
    B = T * Bt
    flat = input.reshape(B).astype(jnp.int32)
    out = _make_gather(V, D, B)(table, flat)
    return out.reshape(T, Bt, D)


# speedup vs baseline: 1.1263x; 1.0172x over previous
import functools

import jax
import jax.numpy as jnp
from jax import lax
from jax.experimental import pallas as pl
from jax.experimental.pallas import tpu as pltpu
from jax.experimental.pallas import tpu_sc as plsc


@functools.lru_cache(maxsize=None)
def _make_gather(V, D, B):
    info = plsc.get_sparse_core_info()
    NC, NS = info.num_cores, info.num_subcores
    NW = NC * NS
    assert B % NW == 0
    b_per_w = B // NW
    C = 1600
    while b_per_w % C != 0:
        C //= 2
    n_chunks = b_per_w // C
    mesh = plsc.VectorSubcoreMesh(core_axis_name="c", subcore_axis_name="s")

    @functools.partial(
        pl.kernel,
        mesh=mesh,
        out_type=jax.ShapeDtypeStruct((B, D), jnp.float32),
        scratch_types=[
            pltpu.VMEM((C,), jnp.int32),
            pltpu.VMEM((C,), jnp.int32),
            pltpu.VMEM((C, D), jnp.float32),
            pltpu.VMEM((C, D), jnp.float32),
            pltpu.SemaphoreType.DMA,
            pltpu.SemaphoreType.DMA,
            pltpu.SemaphoreType.DMA,
            pltpu.SemaphoreType.DMA,
            pltpu.SemaphoreType.DMA,
            pltpu.SemaphoreType.DMA,
        ],
        compiler_params=pltpu.CompilerParams(use_tc_tiling_on_sc=False),
    )
    def gather(
        table_hbm, idx_hbm, out_hbm,
        idx_v0, idx_v1, rows_v0, rows_v1,
        isem0, isem1, gsem0, gsem1, wsem0, wsem1,
    ):
        idx_v = (idx_v0, idx_v1)
        rows_v = (rows_v0, rows_v1)
        isem = (isem0, isem1)
        gsem = (gsem0, gsem1)
        wsem = (wsem0, wsem1)
        wid = lax.axis_index("s") * NC + lax.axis_index("c")
        base = wid * b_per_w

        def start_idx(g):
            s = g % 2
            pltpu.async_copy(
                idx_hbm.at[pl.ds(base + g * C, C)], idx_v[s], isem[s]
            )

        def start_gather(g):
            s = g % 2
            pltpu.async_copy(table_hbm.at[idx_v[s]], rows_v[s], gsem[s])

        def start_wb(g):
            s = g % 2
            pltpu.async_copy(
                rows_v[s], out_hbm.at[pl.ds(base + g * C, C)], wsem[s]
            )

        def wait_idx(g):
            s = g % 2
            pltpu.make_async_copy(
                idx_hbm.at[pl.ds(base + g * C, C)], idx_v[s], isem[s]
            ).wait()

        def wait_gather(g):
            s = g % 2
            pltpu.make_async_copy(table_hbm.at[idx_v[s]], rows_v[s], gsem[s]).wait()

        def wait_wb(g):
            s = g % 2
            pltpu.make_async_copy(
                rows_v[s], out_hbm.at[pl.ds(base + g * C, C)], wsem[s]
            ).wait()

        start_idx(0)
        if n_chunks > 1:
            start_idx(1)
        wait_idx(0)
        start_gather(0)
        for g in range(n_chunks):
            if g + 1 < n_chunks:
                wait_idx(g + 1)
                if g + 1 >= 2:
                    wait_wb(g - 1)
                start_gather(g + 1)
            wait_gather(g)
            if g + 2 < n_chunks:
                start_idx(g + 2)
            start_wb(g)
        for g in (n_chunks - 2, n_chunks - 1):
            if g >= 0:
                wait_wb(g)

    return gather


def kernel(input, table):
    T, Bt = input.shape
    V, D = table.shape
    B = T * Bt
    flat = input.reshape(B).astype(jnp.int32)
    out = _make_gather(V, D, B)(table, flat)
    return out.reshape(T, Bt, D)
